# Initial kernel scaffold; baseline (speedup 1.0000x reference)
import jax
import jax.numpy as jnp
from jax.experimental import pallas as pl


def kernel(input, r):
    raise NotImplementedError("write your pallas kernel here")


# speedup vs baseline: 8.3486x; 8.3486x over previous
import jax
import jax.numpy as jnp
from jax.experimental import pallas as pl
from jax.experimental.pallas import tpu as pltpu

_N = 32768
_ROWS_PER_BLOCK = 8
_EPS = 1e-8


def _f32_to_key(x):
    i = jax.lax.bitcast_convert_type(x, jnp.int32)
    return jnp.where(i < 0, i ^ jnp.int32(0x7FFFFFFF), i)


def _key_to_f32(k):
    i = jnp.where(k < 0, k ^ jnp.int32(0x7FFFFFFF), k)
    return jax.lax.bitcast_convert_type(i, jnp.float32)


def _rsoftmax_block(x_ref, r_ref, o_ref, e_ref, key_ref):
    x = x_ref[...]
    m = jnp.max(x, axis=1, keepdims=True)
    im = x - m
    e = jnp.exp(im)
    e_ref[...] = e
    zmask = e == 0.0
    zf = jnp.sum(zmask.astype(jnp.float32), axis=1, keepdims=True) * (
        1.0 / _N
    )
    r = r_ref[...]
    q = jnp.clip((r - zf) / (1.0 - zf), 0.0, 1.0)
    idx = q * jnp.float32(_N - 1)
    kf = jnp.floor(idx)
    frac = idx - kf
    k = kf.astype(jnp.int32)

    xmm = im * (1.0 - zmask.astype(jnp.float32))
    key_ref[...] = _f32_to_key(xmm)

    cnt_neg = jnp.sum(
        (key_ref[...] < 0).astype(jnp.int32), axis=1, keepdims=True
    )
    ps0 = jnp.where(
        cnt_neg <= k, jnp.zeros_like(k), jnp.full_like(k, jnp.int32(-2147483648))
    )

    def step(i, ps):
        cand = ps | (jnp.int32(1) << (jnp.int32(30) - i))
        cnt = jnp.sum(
            (key_ref[...] < cand).astype(jnp.int32), axis=1, keepdims=True
        )
        return jnp.where(cnt <= k, cand, ps)

    ps = jax.lax.fori_loop(0, 31, step, ps0)
    a_low = _key_to_f32(ps)

    keys = key_ref[...]
    le = keys <= ps
    cnt_le = jnp.sum(le.astype(jnp.int32), axis=1, keepdims=True)
    gmin = jnp.min(
        jnp.where(le, jnp.int32(2147483647), keys), axis=1, keepdims=True
    )
    a_high = _key_to_f32(jnp.where(cnt_le >= k + 2, ps, gmin))
    a_high = jnp.where(frac > 0.0, a_high, a_low)

    quant = a_low * (1.0 - frac) + a_high * frac
    t = _EPS - quant
    num = e_ref[...] * jnp.maximum(im + t, 0.0)
    denom = jnp.sum(num, axis=1, keepdims=True)
    o_ref[...] = num / denom


@jax.jit
def _rsoftmax(x, r):
    grid = (x.shape[0] // _ROWS_PER_BLOCK,)
    return pl.pallas_call(
        _rsoftmax_block,
        grid=grid,
        in_specs=[
            pl.BlockSpec((_ROWS_PER_BLOCK, _N), lambda i: (i, 0)),
            pl.BlockSpec((_ROWS_PER_BLOCK, 1), lambda i: (i, 0)),
        ],
        out_specs=pl.BlockSpec((_ROWS_PER_BLOCK, _N), lambda i: (i, 0)),
        out_shape=jax.ShapeDtypeStruct(x.shape, jnp.float32),
        scratch_shapes=[
            pltpu.VMEM((_ROWS_PER_BLOCK, _N), jnp.float32),
            pltpu.VMEM((_ROWS_PER_BLOCK, _N), jnp.int32),
        ],
    )(x, r)


def kernel(input, r):
    return _rsoftmax(input, r)


# speedup vs baseline: 13.1787x; 1.5785x over previous
import jax
import jax.numpy as jnp
from jax.experimental import pallas as pl
from jax.experimental.pallas import tpu as pltpu

_N = 32768
_ROWS_PER_BLOCK = 16
_EPS = 1e-8
_INT_MIN = -2147483648


def _f32_to_key(x):
    i = jax.lax.bitcast_convert_type(x, jnp.int32)
    return jnp.where(i < 0, i ^ jnp.int32(0x7FFFFFFF), i)


def _key_to_f32(k):
    i = jnp.where(k < 0, k ^ jnp.int32(0x7FFFFFFF), k)
    return jax.lax.bitcast_convert_type(i, jnp.float32)


def _rsoftmax_block(x_ref, r_ref, o_ref, key_ref):
    x = x_ref[...]
    m = jnp.max(x, axis=1, keepdims=True)
    im = x - m
    zmask = jnp.exp(im) == 0.0
    zf = jnp.sum(zmask, axis=1, keepdims=True).astype(jnp.float32) * (
        1.0 / _N
    )
    r = r_ref[...]
    q = jnp.clip((r - zf) / (1.0 - zf), 0.0, 1.0)
    idx = q * jnp.float32(_N - 1)
    kf = jnp.floor(idx)
    frac = idx - kf
    k = kf.astype(jnp.int32)

    key_ref[...] = _f32_to_key(im * (1.0 - zmask.astype(jnp.float32)))

    def step(i, ps):
        cand = ps + (jnp.int32(1) << (jnp.int32(31) - i))
        cnt = jnp.sum(key_ref[...] < cand, axis=1, keepdims=True)
        return jnp.where(cnt <= k, cand, ps)

    ps = jax.lax.fori_loop(
        0, 32, step, jnp.full_like(k, jnp.int32(_INT_MIN))
    )
    a_low = _key_to_f32(ps)

    keys = key_ref[...]
    le = keys <= ps
    cnt_le = jnp.sum(le, axis=1, keepdims=True)
    gmin = jnp.min(
        jnp.where(le, jnp.int32(2147483647), keys), axis=1, keepdims=True
    )
    a_high = _key_to_f32(jnp.where(cnt_le >= k + 2, ps, gmin))
    a_high = jnp.where(frac > 0.0, a_high, a_low)

    quant = a_low * (1.0 - frac) + a_high * frac
    t = _EPS - quant

    im2 = x_ref[...] - m
    num = jnp.exp(im2) * jnp.maximum(im2 + t, 0.0)
    o_ref[...] = num
    denom = jnp.sum(num, axis=1, keepdims=True)
    o_ref[...] = o_ref[...] * (1.0 / denom)


@jax.jit
def _rsoftmax(x, r):
    grid = (x.shape[0] // _ROWS_PER_BLOCK,)
    return pl.pallas_call(
        _rsoftmax_block,
        grid=grid,
        in_specs=[
            pl.BlockSpec((_ROWS_PER_BLOCK, _N), lambda i: (i, 0)),
            pl.BlockSpec((_ROWS_PER_BLOCK, 1), lambda i: (i, 0)),
        ],
        out_specs=pl.BlockSpec((_ROWS_PER_BLOCK, _N), lambda i: (i, 0)),
        out_shape=jax.ShapeDtypeStruct(x.shape, jnp.float32),
        scratch_shapes=[
            pltpu.VMEM((_ROWS_PER_BLOCK, _N), jnp.int32),
        ],
    )(x, r)


def kernel(input, r):
    return _rsoftmax(input, r)


# speedup vs baseline: 13.1820x; 1.0003x over previous
import jax
import jax.numpy as jnp
from jax.experimental import pallas as pl
from jax.experimental.pallas import tpu as pltpu

_N = 32768
_ROWS_PER_BLOCK = 16
_EPS = 1e-8
_INT_MIN = -2147483648


def _f32_to_key(x):
    i = jax.lax.bitcast_convert_type(x, jnp.int32)
    return jnp.where(i < 0, i ^ jnp.int32(0x7FFFFFFF), i)


def _key_to_f32(k):
    i = jnp.where(k < 0, k ^ jnp.int32(0x7FFFFFFF), k)
    return jax.lax.bitcast_convert_type(i, jnp.float32)


def _rsoftmax_block(x_ref, r_ref, o_ref, key_ref):
    x = x_ref[...]
    m = jnp.max(x, axis=1, keepdims=True)
    im = x - m
    zmask = jnp.exp(im) == 0.0
    zf = jnp.sum(zmask, axis=1, keepdims=True).astype(jnp.float32) * (
        1.0 / _N
    )
    r = r_ref[...]
    q = jnp.clip((r - zf) / (1.0 - zf), 0.0, 1.0)
    idx = q * jnp.float32(_N - 1)
    kf = jnp.floor(idx)
    frac = idx - kf
    k = kf.astype(jnp.int32)

    key_ref[...] = _f32_to_key(im * (1.0 - zmask.astype(jnp.float32)))

    def step(i, ps):
        cand = ps + (jnp.int32(1) << (jnp.int32(31) - i))
        cnt = jnp.sum(key_ref[...] < cand, axis=1, keepdims=True)
        return jnp.where(cnt <= k, cand, ps)

    ps = jax.lax.fori_loop(
        0, 32, step, jnp.full_like(k, jnp.int32(_INT_MIN))
    )
    a_low = _key_to_f32(ps)

    keys = key_ref[...]
    le = keys <= ps
    cnt_le = jnp.sum(le, axis=1, keepdims=True)
    gmin = jnp.min(
        jnp.where(le, jnp.int32(2147483647), keys), axis=1, keepdims=True
    )
    a_high = _key_to_f32(jnp.where(cnt_le >= k + 2, ps, gmin))
    a_high = jnp.where(frac > 0.0, a_high, a_low)

    quant = a_low * (1.0 - frac) + a_high * frac
    t = _EPS - quant

    im2 = x_ref[...] - m
    num = jnp.exp(im2) * jnp.maximum(im2 + t, 0.0)
    o_ref[...] = num
    denom = jnp.sum(num, axis=1, keepdims=True)
    o_ref[...] = o_ref[...] * (1.0 / denom)


@jax.jit
def _rsoftmax(x, r):
    grid = (x.shape[0] // _ROWS_PER_BLOCK,)
    return pl.pallas_call(
        _rsoftmax_block,
        grid=grid,
        in_specs=[
            pl.BlockSpec((_ROWS_PER_BLOCK, _N), lambda i: (i, 0)),
            pl.BlockSpec((_ROWS_PER_BLOCK, 1), lambda i: (i, 0)),
        ],
        out_specs=pl.BlockSpec((_ROWS_PER_BLOCK, _N), lambda i: (i, 0)),
        out_shape=jax.ShapeDtypeStruct(x.shape, jnp.float32),
        scratch_shapes=[
            pltpu.VMEM((_ROWS_PER_BLOCK, _N), jnp.int32),
        ],
        compiler_params=pltpu.CompilerParams(
            dimension_semantics=("parallel",),
        ),
    )(x, r)


def kernel(input, r):
    return _rsoftmax(input, r)


# speedup vs baseline: 17.1188x; 1.2987x over previous
import jax
import jax.numpy as jnp
from jax.experimental import pallas as pl
from jax.experimental.pallas import tpu as pltpu

_N = 32768
_ROWS_PER_BLOCK = 32
_EPS = 1e-8
_INT_MIN = -2147483648


def _f32_to_key(x):
    i = jax.lax.bitcast_convert_type(x, jnp.int32)
    return jnp.where(i < 0, i ^ jnp.int32(0x7FFFFFFF), i)


def _key_to_f32(k):
    i = jnp.where(k < 0, k ^ jnp.int32(0x7FFFFFFF), k)
    return jax.lax.bitcast_convert_type(i, jnp.float32)


def _rsoftmax_block(x_ref, r_ref, o_ref, key_ref):
    x = x_ref[...]
    m = jnp.max(x, axis=1, keepdims=True)
    im = x - m
    zmask = jnp.exp(im) == 0.0
    zf = jnp.sum(zmask, axis=1, keepdims=True).astype(jnp.float32) * (
        1.0 / _N
    )
    r = r_ref[...]
    q = jnp.clip((r - zf) / (1.0 - zf), 0.0, 1.0)
    idx = q * jnp.float32(_N - 1)
    kf = jnp.floor(idx)
    frac = idx - kf
    k = kf.astype(jnp.int32)

    key_ref[...] = _f32_to_key(im * (1.0 - zmask.astype(jnp.float32)))

    def step(i, ps):
        cand = ps + (jnp.int32(1) << (jnp.int32(31) - i))
        cnt = jnp.sum(key_ref[...] < cand, axis=1, keepdims=True)
        return jnp.where(cnt <= k, cand, ps)

    ps = jax.lax.fori_loop(
        0, 32, step, jnp.full_like(k, jnp.int32(_INT_MIN))
    )
    a_low = _key_to_f32(ps)

    keys = key_ref[...]
    le = keys <= ps
    cnt_le = jnp.sum(le, axis=1, keepdims=True)
    gmin = jnp.min(
        jnp.where(le, jnp.int32(2147483647), keys), axis=1, keepdims=True
    )
    a_high = _key_to_f32(jnp.where(cnt_le >= k + 2, ps, gmin))
    a_high = jnp.where(frac > 0.0, a_high, a_low)

    quant = a_low * (1.0 - frac) + a_high * frac
    t = _EPS - quant

    im2 = x_ref[...] - m
    num = jnp.exp(im2) * jnp.maximum(im2 + t, 0.0)
    o_ref[...] = num
    denom = jnp.sum(num, axis=1, keepdims=True)
    o_ref[...] = o_ref[...] * (1.0 / denom)


@jax.jit
def _rsoftmax(x, r):
    grid = (x.shape[0] // _ROWS_PER_BLOCK,)
    return pl.pallas_call(
        _rsoftmax_block,
        grid=grid,
        in_specs=[
            pl.BlockSpec((_ROWS_PER_BLOCK, _N), lambda i: (i, 0)),
            pl.BlockSpec((_ROWS_PER_BLOCK, 1), lambda i: (i, 0)),
        ],
        out_specs=pl.BlockSpec((_ROWS_PER_BLOCK, _N), lambda i: (i, 0)),
        out_shape=jax.ShapeDtypeStruct(x.shape, jnp.float32),
        scratch_shapes=[
            pltpu.VMEM((_ROWS_PER_BLOCK, _N), jnp.int32),
        ],
        compiler_params=pltpu.CompilerParams(
            dimension_semantics=("parallel",),
        ),
    )(x, r)


def kernel(input, r):
    return _rsoftmax(input, r)


# speedup vs baseline: 19.9922x; 1.1679x over previous
import jax
import jax.numpy as jnp
from jax.experimental import pallas as pl
from jax.experimental.pallas import tpu as pltpu

_N = 32768
_ROWS_PER_BLOCK = 64
_EPS = 1e-8
_INT_MIN = -2147483648


def _f32_to_key(x):
    i = jax.lax.bitcast_convert_type(x, jnp.int32)
    return jnp.where(i < 0, i ^ jnp.int32(0x7FFFFFFF), i)


def _key_to_f32(k):
    i = jnp.where(k < 0, k ^ jnp.int32(0x7FFFFFFF), k)
    return jax.lax.bitcast_convert_type(i, jnp.float32)


def _rsoftmax_block(x_ref, r_ref, o_ref, key_ref):
    x = x_ref[...]
    m = jnp.max(x, axis=1, keepdims=True)
    im = x - m
    zmask = jnp.exp(im) == 0.0
    zf = jnp.sum(zmask, axis=1, keepdims=True).astype(jnp.float32) * (
        1.0 / _N
    )
    r = r_ref[...]
    q = jnp.clip((r - zf) / (1.0 - zf), 0.0, 1.0)
    idx = q * jnp.float32(_N - 1)
    kf = jnp.floor(idx)
    frac = idx - kf
    k = kf.astype(jnp.int32)

    key_ref[...] = _f32_to_key(im * (1.0 - zmask.astype(jnp.float32)))

    def step(i, ps):
        cand = ps + (jnp.int32(1) << (jnp.int32(31) - i))
        cnt = jnp.sum(key_ref[...] < cand, axis=1, keepdims=True)
        return jnp.where(cnt <= k, cand, ps)

    ps = jax.lax.fori_loop(
        0, 32, step, jnp.full_like(k, jnp.int32(_INT_MIN))
    )
    a_low = _key_to_f32(ps)

    keys = key_ref[...]
    le = keys <= ps
    cnt_le = jnp.sum(le, axis=1, keepdims=True)
    gmin = jnp.min(
        jnp.where(le, jnp.int32(2147483647), keys), axis=1, keepdims=True
    )
    a_high = _key_to_f32(jnp.where(cnt_le >= k + 2, ps, gmin))
    a_high = jnp.where(frac > 0.0, a_high, a_low)

    quant = a_low * (1.0 - frac) + a_high * frac
    t = _EPS - quant

    im2 = x_ref[...] - m
    num = jnp.exp(im2) * jnp.maximum(im2 + t, 0.0)
    o_ref[...] = num
    denom = jnp.sum(num, axis=1, keepdims=True)
    o_ref[...] = o_ref[...] * (1.0 / denom)


@jax.jit
def _rsoftmax(x, r):
    grid = (x.shape[0] // _ROWS_PER_BLOCK,)
    return pl.pallas_call(
        _rsoftmax_block,
        grid=grid,
        in_specs=[
            pl.BlockSpec((_ROWS_PER_BLOCK, _N), lambda i: (i, 0)),
            pl.BlockSpec((_ROWS_PER_BLOCK, 1), lambda i: (i, 0)),
        ],
        out_specs=pl.BlockSpec((_ROWS_PER_BLOCK, _N), lambda i: (i, 0)),
        out_shape=jax.ShapeDtypeStruct(x.shape, jnp.float32),
        scratch_shapes=[
            pltpu.VMEM((_ROWS_PER_BLOCK, _N), jnp.int32),
        ],
        compiler_params=pltpu.CompilerParams(
            dimension_semantics=("parallel",),
        ),
    )(x, r)


def kernel(input, r):
    return _rsoftmax(input, r)
